# baseline (speedup 1.0000x reference)
import functools

import jax
import jax.numpy as jnp
from jax import lax
from jax.experimental import pallas as pl
from jax.experimental.pallas import tpu as pltpu
from jax.experimental.pallas import tpu_sc as plsc

_NUM_ENT = 1000000
_NUM_REL = 1000
_D = 64
_DH = 32
_B = 16384

_NW = 32
_BPW = _B // _NW
_CHUNK = 128
_NCH = _BPW // _CHUNK
_G = _BPW // 16


def _transe_body(hs_hbm, rs_hbm, ts_hbm, ent_lo, ent_hi, rel_lo, rel_hi,
                 out_hbm, hs_v, rs_v, ts_v,
                 h_lo, h_hi, r_lo, r_hi, t_lo, t_hi, o_v, sem):
    wid = lax.axis_index("s") * 2 + lax.axis_index("c")
    base = wid * _BPW

    for c in range(_NCH):
        off = base + c * _CHUNK
        pltpu.sync_copy(hs_hbm.at[pl.ds(off, _CHUNK)], hs_v.at[c])
        pltpu.sync_copy(rs_hbm.at[pl.ds(off, _CHUNK)], rs_v.at[c])
        pltpu.sync_copy(ts_hbm.at[pl.ds(off, _CHUNK)], ts_v.at[c])

    copies = []
    for c in range(_NCH):
        dst = pl.ds(c * _CHUNK, _CHUNK)
        copies.append(pltpu.async_copy(ent_lo.at[hs_v.at[c]], h_lo.at[dst], sem))
        copies.append(pltpu.async_copy(ent_hi.at[hs_v.at[c]], h_hi.at[dst], sem))
        copies.append(pltpu.async_copy(rel_lo.at[rs_v.at[c]], r_lo.at[dst], sem))
        copies.append(pltpu.async_copy(rel_hi.at[rs_v.at[c]], r_hi.at[dst], sem))
        copies.append(pltpu.async_copy(ent_lo.at[ts_v.at[c]], t_lo.at[dst], sem))
        copies.append(pltpu.async_copy(ent_hi.at[ts_v.at[c]], t_hi.at[dst], sem))
    for cp in copies:
        cp.wait()

    iota16 = lax.iota(jnp.int32, 16)
    half = jnp.float32(0.5)
    threehalf = jnp.float32(1.5)
    magic = jnp.int32(0x5F3759DF)

    def group_body(g, carry):
        rows = g * 16 + iota16

        def j_body(j, accs):
            a0, a1, a2, a3 = accs
            c0 = jnp.full((16,), 2 * j, jnp.int32)
            c1 = jnp.full((16,), 2 * j + 1, jnp.int32)
            d0 = (plsc.load_gather(h_lo, [rows, c0])
                  + plsc.load_gather(r_lo, [rows, c0])
                  - plsc.load_gather(t_lo, [rows, c0]))
            d1 = (plsc.load_gather(h_lo, [rows, c1])
                  + plsc.load_gather(r_lo, [rows, c1])
                  - plsc.load_gather(t_lo, [rows, c1]))
            d2 = (plsc.load_gather(h_hi, [rows, c0])
                  + plsc.load_gather(r_hi, [rows, c0])
                  - plsc.load_gather(t_hi, [rows, c0]))
            d3 = (plsc.load_gather(h_hi, [rows, c1])
                  + plsc.load_gather(r_hi, [rows, c1])
                  - plsc.load_gather(t_hi, [rows, c1]))
            return (a0 + d0 * d0, a1 + d1 * d1, a2 + d2 * d2, a3 + d3 * d3)

        z = jnp.zeros((16,), jnp.float32)
        a0, a1, a2, a3 = lax.fori_loop(0, _DH // 2, j_body, (z, z, z, z))
        acc = (a0 + a1) + (a2 + a3)

        bits = lax.bitcast_convert_type(acc, jnp.int32)
        y = lax.bitcast_convert_type(magic - (bits >> 1), jnp.float32)
        hx = half * acc
        for _ in range(3):
            y = y * (threehalf - hx * y * y)
        o_v[pl.ds(g * 16, 16)] = acc * y
        return carry

    lax.fori_loop(0, _G, group_body, jnp.int32(0))
    pltpu.sync_copy(o_v, out_hbm.at[pl.ds(base, _BPW)])


@jax.jit
def _transe_call(hs, rs, ts, ent_lo, ent_hi, rel_lo, rel_hi):
    mesh = plsc.VectorSubcoreMesh(core_axis_name="c", subcore_axis_name="s")
    fn = functools.partial(
        pl.kernel,
        mesh=mesh,
        out_type=jax.ShapeDtypeStruct((_B,), jnp.float32),
        compiler_params=pltpu.CompilerParams(
            use_tc_tiling_on_sc=False, needs_layout_passes=False
        ),
        scratch_types=[
            pltpu.VMEM((_NCH, _CHUNK), jnp.int32),
            pltpu.VMEM((_NCH, _CHUNK), jnp.int32),
            pltpu.VMEM((_NCH, _CHUNK), jnp.int32),
            pltpu.VMEM((_BPW, _DH), jnp.float32),
            pltpu.VMEM((_BPW, _DH), jnp.float32),
            pltpu.VMEM((_BPW, _DH), jnp.float32),
            pltpu.VMEM((_BPW, _DH), jnp.float32),
            pltpu.VMEM((_BPW, _DH), jnp.float32),
            pltpu.VMEM((_BPW, _DH), jnp.float32),
            pltpu.VMEM((_BPW,), jnp.float32),
            pltpu.SemaphoreType.DMA,
        ],
    )(_transe_body)
    return fn(hs, rs, ts, ent_lo, ent_hi, rel_lo, rel_hi)


def kernel(hs, rs, ts, ent_embs, rel_embs):
    out = _transe_call(hs, rs, ts,
                       ent_embs[:, :_DH], ent_embs[:, _DH:],
                       rel_embs[:, :_DH], rel_embs[:, _DH:])
    return out.reshape(-1, 1)


# speedup vs baseline: 2.1839x; 2.1839x over previous
import functools

import jax
import jax.numpy as jnp
from jax import lax
from jax.experimental import pallas as pl
from jax.experimental.pallas import tpu as pltpu
from jax.experimental.pallas import tpu_sc as plsc

_NUM_ENT = 1000000
_NUM_REL = 1000
_D = 64
_B = 16384

_NW = 32
_BPW = _B // _NW
_CHUNK = 128
_NCH = _BPW // _CHUNK
_G = _BPW // 16


def _transe_body(hs_hbm, rs_hbm, ts_hbm, ent_hbm, rel_hbm, out_hbm,
                 hs_v, rs_v, ts_v, h_v, r_v, t_v, o_v, sem):
    wid = lax.axis_index("s") * 2 + lax.axis_index("c")
    base = wid * _BPW

    for c in range(_NCH):
        off = base + c * _CHUNK
        pltpu.sync_copy(hs_hbm.at[pl.ds(off, _CHUNK)], hs_v.at[c])
        pltpu.sync_copy(rs_hbm.at[pl.ds(off, _CHUNK)], rs_v.at[c])
        pltpu.sync_copy(ts_hbm.at[pl.ds(off, _CHUNK)], ts_v.at[c])

    copies = []
    for c in range(_NCH):
        dst = pl.ds(c * _CHUNK, _CHUNK)
        copies.append(pltpu.async_copy(ent_hbm.at[hs_v.at[c]], h_v.at[dst], sem))
        copies.append(pltpu.async_copy(rel_hbm.at[rs_v.at[c]], r_v.at[dst], sem))
        copies.append(pltpu.async_copy(ent_hbm.at[ts_v.at[c]], t_v.at[dst], sem))
    for cp in copies:
        cp.wait()

    iota16 = lax.iota(jnp.int32, 16)
    half = jnp.float32(0.5)
    threehalf = jnp.float32(1.5)
    magic = jnp.int32(0x5F3759DF)

    def group_body(g, carry):
        rows = g * 16 + iota16

        def j_body(j, accs):
            a0, a1, a2, a3 = accs
            ds = []
            for q in range(4):
                cq = jnp.full((16,), 4 * j + q, jnp.int32)
                ds.append(plsc.load_gather(h_v, [rows, cq])
                          + plsc.load_gather(r_v, [rows, cq])
                          - plsc.load_gather(t_v, [rows, cq]))
            return (a0 + ds[0] * ds[0], a1 + ds[1] * ds[1],
                    a2 + ds[2] * ds[2], a3 + ds[3] * ds[3])

        z = jnp.zeros((16,), jnp.float32)
        a0, a1, a2, a3 = lax.fori_loop(0, _D // 4, j_body, (z, z, z, z))
        acc = (a0 + a1) + (a2 + a3)

        bits = lax.bitcast_convert_type(acc, jnp.int32)
        y = lax.bitcast_convert_type(magic - (bits >> 1), jnp.float32)
        hx = half * acc
        for _ in range(3):
            y = y * (threehalf - hx * y * y)
        o_v[pl.ds(g * 16, 16)] = acc * y
        return carry

    lax.fori_loop(0, _G, group_body, jnp.int32(0))
    pltpu.sync_copy(o_v, out_hbm.at[pl.ds(base, _BPW)])


@jax.jit
def _transe_call(hs, rs, ts, ent_embs, rel_embs):
    mesh = plsc.VectorSubcoreMesh(core_axis_name="c", subcore_axis_name="s")
    fn = functools.partial(
        pl.kernel,
        mesh=mesh,
        out_type=jax.ShapeDtypeStruct((_B,), jnp.float32),
        compiler_params=pltpu.CompilerParams(
            use_tc_tiling_on_sc=False, needs_layout_passes=False
        ),
        scratch_types=[
            pltpu.VMEM((_NCH, _CHUNK), jnp.int32),
            pltpu.VMEM((_NCH, _CHUNK), jnp.int32),
            pltpu.VMEM((_NCH, _CHUNK), jnp.int32),
            pltpu.VMEM((_BPW, _D), jnp.float32),
            pltpu.VMEM((_BPW, _D), jnp.float32),
            pltpu.VMEM((_BPW, _D), jnp.float32),
            pltpu.VMEM((_BPW,), jnp.float32),
            pltpu.SemaphoreType.DMA,
        ],
    )(_transe_body)
    return fn(hs, rs, ts, ent_embs, rel_embs)


def kernel(hs, rs, ts, ent_embs, rel_embs):
    out = _transe_call(hs, rs, ts, ent_embs, rel_embs)
    return out.reshape(-1, 1)
